# baseline (speedup 1.0000x reference)
import functools

import jax
import jax.numpy as jnp
from jax import lax
from jax.experimental import pallas as pl
from jax.experimental.pallas import tpu as pltpu
from jax.experimental.pallas import tpu_sc as plsc

N = 50000
NP = 51200
E = 800000
EP = 819200
ECH = 256
HQ = 16
NSUB = 16
NIDX = EP // NSUB
GCH = 8
GW = GCH * ECH
NG = NIDX // GW
NRING = 4
RT = NP // NSUB
ZR = 400
SR = 800
NGRAPH = 64

_mesh = plsc.VectorSubcoreMesh(core_axis_name="c", subcore_axis_name="s")
_SC_PARAMS = pltpu.CompilerParams(use_tc_tiling_on_sc=False,
                                  needs_layout_passes=False)

_QSHAPE = jax.ShapeDtypeStruct((NP, HQ), jnp.float32)


@functools.partial(
    pl.kernel,
    out_type=jax.ShapeDtypeStruct((4, NP, HQ), jnp.float32),
    mesh=_mesh,
    compiler_params=_SC_PARAMS,
    scratch_types=[
        pltpu.VMEM((2, GW), jnp.int32),
        pltpu.VMEM((2, GW), jnp.int32),
        pltpu.VMEM((NRING, ECH, HQ), jnp.float32),
        pltpu.VMEM((ZR, HQ), jnp.float32),
        pltpu.VMEM((ECH, HQ), jnp.float32),
        pltpu.VMEM((16,), jnp.int32),
        pltpu.VMEM_SHARED((NP, HQ), jnp.float32),
        pltpu.SemaphoreType.DMA((NRING,)),
        pltpu.SemaphoreType.DMA((2,)),
    ],
)
def _scatter_kernel(hst_hbm, src_hbm, dst_hbm, z16_hbm,
                    ones_hbm, fl_hbm, ast_hbm,
                    sbuf, dbuf, rowsv, zv, onesv, modev, accsp, gsem, isem):
    c = lax.axis_index("c")
    s = lax.axis_index("s")
    ibase = s * NIDX

    def fire_idx(g, k):
        pltpu.async_copy(src_hbm.at[pl.ds(ibase + g * GW, GW)],
                         sbuf.at[k], isem.at[k])
        pltpu.async_copy(dst_hbm.at[pl.ds(ibase + g * GW, GW)],
                         dbuf.at[k], isem.at[k])

    def wait_idx(k):
        pltpu.make_async_copy(src_hbm.at[pl.ds(0, GW)], sbuf.at[k],
                              isem.at[k]).wait()
        pltpu.make_async_copy(dst_hbm.at[pl.ds(0, GW)], dbuf.at[k],
                              isem.at[k]).wait()

    pltpu.sync_copy(z16_hbm, zv)
    pltpu.sync_copy(ones_hbm, onesv)
    pltpu.sync_copy(fl_hbm, modev)
    is_deg = modev[pl.ds(0, 16)][0] == 1

    def zero_acc():
        for k in range(8):
            pltpu.sync_copy(zv, accsp.at[pl.ds(s * RT + k * ZR, ZR)])
        plsc.subcore_barrier()

    def writeback(out_hbm):
        plsc.subcore_barrier()
        for k in range(8):
            r0 = s * RT + k * ZR
            pltpu.sync_copy(accsp.at[pl.ds(r0, ZR)], zv)
            pltpu.sync_copy(zv, out_hbm.at[pl.ds(r0, ZR)])
        plsc.subcore_barrier()
        pltpu.sync_copy(z16_hbm, zv)

    def fire_gather(hs_hbm, k, g, j):
        pltpu.async_copy(hs_hbm.at[sbuf.at[k, pl.ds(j * ECH, ECH)]],
                         rowsv.at[(g * GCH + j) % NRING],
                         gsem.at[(g * GCH + j) % NRING])

    def run(hs_hbm, out_hbm):
        zero_acc()
        fire_idx(0, 0)
        wait_idx(0)
        for j in range(NRING - 1):
            fire_gather(hs_hbm, 0, 0, j)

        def group(g, carry):
            k = lax.rem(g, 2)
            kn = lax.rem(g + 1, 2)

            @pl.when(g + 1 < NG)
            def _pf_idx():
                fire_idx(g + 1, kn)

            for j in range(GCH):
                kr = j % NRING
                pltpu.make_async_copy(
                    hs_hbm.at[sbuf.at[k, pl.ds(j * ECH, ECH)]],
                    rowsv.at[kr], gsem.at[kr]).wait()
                if j == 2:
                    @pl.when(g + 1 < NG)
                    def _w_idx():
                        wait_idx(kn)
                nxt = j + NRING - 1
                if nxt < GCH:
                    fire_gather(hs_hbm, k, g, nxt)
                else:
                    @pl.when(g + 1 < NG)
                    def _pf_gather():
                        fire_gather(hs_hbm, kn, g + 1, nxt - GCH)
                pltpu.sync_copy(rowsv.at[kr],
                                accsp.at[dbuf.at[k, pl.ds(j * ECH, ECH)]],
                                add=True)
            return carry

        lax.fori_loop(0, NG, group, 0)
        writeback(out_hbm)

    def run_deg(out_hbm):
        zero_acc()
        fire_idx(0, 0)

        def group(g, carry):
            k = lax.rem(g, 2)
            kn = lax.rem(g + 1, 2)
            wait_idx(k)

            @pl.when(g + 1 < NG)
            def _pf_idx():
                fire_idx(g + 1, kn)

            for j in range(GCH):
                pltpu.sync_copy(onesv,
                                accsp.at[dbuf.at[k, pl.ds(j * ECH, ECH)]],
                                add=True)
            return carry

        lax.fori_loop(0, NG, group, 0)
        writeback(out_hbm)

    @pl.when(jnp.logical_and(c == 0, is_deg))
    def _deg():
        run_deg(ast_hbm.at[0])

    @pl.when(jnp.logical_and(c == 0, jnp.logical_not(is_deg)))
    def _c0():
        run(hst_hbm.at[0], ast_hbm.at[0])
        run(hst_hbm.at[1], ast_hbm.at[1])

    @pl.when(jnp.logical_and(c == 1, jnp.logical_not(is_deg)))
    def _c1():
        run(hst_hbm.at[2], ast_hbm.at[2])
        run(hst_hbm.at[3], ast_hbm.at[3])


_PSHAPE = jax.ShapeDtypeStruct((NSUB, NGRAPH, HQ), jnp.float32)


@functools.partial(
    pl.kernel,
    out_type=jax.ShapeDtypeStruct((4, NSUB, NGRAPH, HQ), jnp.float32),
    mesh=_mesh,
    compiler_params=_SC_PARAMS,
    scratch_types=[
        pltpu.VMEM((SR, HQ), jnp.float32),
        pltpu.VMEM((RT + 16,), jnp.int32),
        pltpu.VMEM((NGRAPH, HQ), jnp.float32),
    ],
)
def _segmax_kernel(hst_hbm, batch_hbm, ost_hbm, hv, bv, tbl):
    c = lax.axis_index("c")
    s = lax.axis_index("s")
    r0 = s * RT
    nvalid = jnp.where(s == NSUB - 1, N - (NSUB - 1) * RT, RT)
    pltpu.sync_copy(batch_hbm.at[pl.ds(r0, RT)], bv.at[pl.ds(0, RT)])

    iota = lax.iota(jnp.int32, 16)
    neginf = jnp.full((16,), -jnp.inf, jnp.float32)

    def run(h_hbm, out_hbm):
        def init(i, carry):
            plsc.store_scatter(tbl, [jnp.full((16,), i, jnp.int32), iota],
                               neginf)
            return carry

        lax.fori_loop(0, NGRAPH, init, 0)

        for ci in range(RT // SR):
            pltpu.sync_copy(h_hbm.at[pl.ds(r0 + ci * SR, SR)], hv)
            nv = jnp.clip(nvalid - ci * SR, 0, SR)

            def step(i, carry):
                b = bv[pl.ds(ci * SR + i, 16)][0]
                bi = jnp.full((16,), b, jnp.int32)
                ii = jnp.full((16,), i, jnp.int32)
                rv = plsc.load_gather(hv, [ii, iota])
                tv = plsc.load_gather(tbl, [bi, iota])
                plsc.store_scatter(tbl, [bi, iota], jnp.maximum(rv, tv))
                return carry

            lax.fori_loop(0, nv, step, 0)
        pltpu.sync_copy(tbl, out_hbm.at[s])

    @pl.when(c == 0)
    def _c0():
        run(hst_hbm.at[0], ost_hbm.at[0])
        run(hst_hbm.at[1], ost_hbm.at[1])

    @pl.when(c == 1)
    def _c1():
        run(hst_hbm.at[2], ost_hbm.at[2])
        run(hst_hbm.at[3], ost_hbm.at[3])


_RB = 2560
_NB = NP // _RB

_PREC = jax.lax.Precision.HIGHEST


def _mid_body(flags_ref, a_ref, s_ref, xq_ref,
              d_ref, b_ref, w_ref, o_ref, dout_ref):
    is_deg = flags_ref[0] == 1
    use_t = flags_ref[1] == 1
    use_relu = flags_ref[2] == 1
    use_scale = flags_ref[3] == 1

    a = a_ref[...]
    sv = s_ref[...]
    b = b_ref[...]
    d = jnp.where(is_deg, lax.rsqrt(jnp.abs(a[0, :, 0:1]) + 1.0),
                  d_ref[...][:, 0:1])
    w = w_ref[...]
    z = None
    for q in range(4):
        base = xq_ref[...] if q == 0 else 0.0
        tq = jnp.where(use_t, d * (a[q] + sv[q]) + b[:, q*HQ:(q+1)*HQ],
                       base)
        hq = jnp.where(use_relu, jnp.maximum(tq, 0.0), tq)
        zq = jnp.dot(hq, w[q*HQ:(q+1)*HQ], precision=_PREC)
        z = zq if z is None else z + zq
    scale = jnp.where(use_scale, d, 1.0)
    zs = scale * z
    o_ref[...] = jnp.stack(
        [zs[:, 0:HQ], zs[:, HQ:2*HQ], zs[:, 2*HQ:3*HQ], zs[:, 3*HQ:]])
    dout_ref[...] = jnp.broadcast_to(d, (_RB, HQ))


def _mid_call(flags, a, s, xq, d, b, W):
    qspec = pl.BlockSpec((4, _RB, HQ), lambda i: (0, i, 0))
    return pl.pallas_call(
        _mid_body,
        grid=(_NB,),
        in_specs=[
            pl.BlockSpec(memory_space=pltpu.SMEM),
            qspec,
            qspec,
            pl.BlockSpec((_RB, HQ), lambda i: (i, 0)),
            pl.BlockSpec((_RB, HQ), lambda i: (i, 0)),
            pl.BlockSpec((1, 64), lambda i: (0, 0)),
            pl.BlockSpec((64, 64), lambda i: (0, 0)),
        ],
        out_specs=[qspec, pl.BlockSpec((_RB, HQ), lambda i: (i, 0))],
        out_shape=[jax.ShapeDtypeStruct((4, NP, HQ), jnp.float32),
                   jax.ShapeDtypeStruct((NP, HQ), jnp.float32)],
    )(flags, a, s, xq, d, b, W)


def _readout_body(p_ref, wl_ref, bl_ref, out_ref):
    wl = wl_ref[...]
    p = p_ref[...]
    z = None
    for q in range(4):
        gq = jnp.max(p[q], axis=0)
        zq = jnp.dot(gq, wl[q*HQ:(q+1)*HQ], precision=_PREC)
        z = zq if z is None else z + zq
    out_ref[...] = z + bl_ref[...]


def _readout_call(p, Wl, bl):
    return pl.pallas_call(
        _readout_body,
        out_shape=jax.ShapeDtypeStruct((NGRAPH, 10), jnp.float32),
    )(p, Wl, bl)


def kernel(x, edge_index, batch, W1, b1, W2, b2, W3, b3, Wl, bl):
    i32 = jnp.int32
    f32 = jnp.float32
    pad = jnp.full((EP - E,), NP - 1, i32)
    src1 = jnp.concatenate([edge_index[0].astype(i32), pad])
    dst1 = jnp.concatenate([edge_index[1].astype(i32), pad])
    xq = jnp.zeros((NP, HQ), f32).at[:N, :x.shape[1]].set(x)
    batchp = jnp.concatenate(
        [batch.astype(i32), jnp.full((NP - N,), NGRAPH - 1, i32)])
    z16 = jnp.zeros((ZR, HQ), f32)

    w1p = jnp.zeros((64, 64), f32).at[:x.shape[1]].set(W1)
    eye = jnp.eye(64, dtype=f32)
    Ws = jnp.stack([w1p, W2, W3, eye])
    bs = jnp.stack([jnp.zeros((1, 64), f32), b1.reshape(1, 64),
                    b2.reshape(1, 64), b3.reshape(1, 64)])
    flags = jnp.array([[1, 0, 0, 1],
                       [0, 1, 1, 1],
                       [0, 1, 1, 1],
                       [0, 1, 0, 0]], dtype=i32)
    flags_sc = jnp.zeros((4, 16), i32).at[0, 0].set(1)
    ones_rows = jnp.ones((ECH, HQ), f32)

    ones_st = jnp.ones((4, NP, HQ), f32)
    d0 = jnp.ones((NP, HQ), f32)

    def body(k, carry):
        hst, d = carry
        fl = lax.dynamic_index_in_dim(flags, k, 0, keepdims=False)
        fls = lax.dynamic_index_in_dim(flags_sc, k, 0, keepdims=False)
        W = lax.dynamic_index_in_dim(Ws, k, 0, keepdims=False)
        b = lax.dynamic_index_in_dim(bs, k, 0, keepdims=False)
        ast = _scatter_kernel(hst, src1, dst1, z16, ones_rows, fls)
        hst, d = _mid_call(fl, ast, hst, xq, d, b, W)
        return (hst, d)

    n_pass = lax.optimization_barrier(jnp.int32(4))
    hst, d = lax.fori_loop(0, n_pass, body, (ones_st, d0))

    p = _segmax_kernel(hst, batchp)
    return _readout_call(p, Wl, bl.reshape(1, 10))


# speedup vs baseline: 1.0277x; 1.0277x over previous
import functools

import jax
import jax.numpy as jnp
from jax import lax
from jax.experimental import pallas as pl
from jax.experimental.pallas import tpu as pltpu
from jax.experimental.pallas import tpu_sc as plsc

N = 50000
NP = 51200
E = 800000
EP = 819200
ECH = 256
HQ = 16
NSUB = 16
NIDX = EP // NSUB
GCH = 8
GW = GCH * ECH
NG = NIDX // GW
NRING = 4
RT = NP // NSUB
ZR = 400
SR = 800
NGRAPH = 64

_mesh = plsc.VectorSubcoreMesh(core_axis_name="c", subcore_axis_name="s")
_SC_PARAMS = pltpu.CompilerParams(use_tc_tiling_on_sc=False,
                                  needs_layout_passes=False)

_QSHAPE = jax.ShapeDtypeStruct((NP, HQ), jnp.float32)


@functools.partial(
    pl.kernel,
    out_type=jax.ShapeDtypeStruct((4, NP, HQ), jnp.float32),
    mesh=_mesh,
    compiler_params=_SC_PARAMS,
    scratch_types=[
        pltpu.VMEM((2, GW), jnp.int32),
        pltpu.VMEM((2, GW), jnp.int32),
        pltpu.VMEM((NRING, ECH, HQ), jnp.float32),
        pltpu.VMEM((ZR, HQ), jnp.float32),
        pltpu.VMEM((ECH, HQ), jnp.float32),
        pltpu.VMEM((16,), jnp.int32),
        pltpu.VMEM_SHARED((NP, HQ), jnp.float32),
        pltpu.SemaphoreType.DMA((NRING,)),
        pltpu.SemaphoreType.DMA((2,)),
    ],
)
def _scatter_kernel(hst_hbm, src_hbm, dst_hbm, z16_hbm,
                    ones_hbm, fl_hbm, ast_hbm,
                    sbuf, dbuf, rowsv, zv, onesv, modev, accsp, gsem, isem):
    c = lax.axis_index("c")
    s = lax.axis_index("s")
    ibase = s * NIDX

    def fire_idx(g, k):
        pltpu.async_copy(src_hbm.at[pl.ds(ibase + g * GW, GW)],
                         sbuf.at[k], isem.at[k])
        pltpu.async_copy(dst_hbm.at[pl.ds(ibase + g * GW, GW)],
                         dbuf.at[k], isem.at[k])

    def wait_idx(k):
        pltpu.make_async_copy(src_hbm.at[pl.ds(0, GW)], sbuf.at[k],
                              isem.at[k]).wait()
        pltpu.make_async_copy(dst_hbm.at[pl.ds(0, GW)], dbuf.at[k],
                              isem.at[k]).wait()

    pltpu.sync_copy(z16_hbm, zv)
    pltpu.sync_copy(ones_hbm, onesv)
    pltpu.sync_copy(fl_hbm, modev)
    is_deg = modev[pl.ds(0, 16)][0] == 1

    def zero_acc():
        for k in range(8):
            pltpu.sync_copy(zv, accsp.at[pl.ds(s * RT + k * ZR, ZR)])
        plsc.subcore_barrier()

    def writeback(out_hbm):
        plsc.subcore_barrier()
        for k in range(8):
            r0 = s * RT + k * ZR
            pltpu.sync_copy(accsp.at[pl.ds(r0, ZR)], zv)
            pltpu.sync_copy(zv, out_hbm.at[pl.ds(r0, ZR)])
        plsc.subcore_barrier()
        pltpu.sync_copy(z16_hbm, zv)

    def fire_gather(hs_hbm, k, g, j):
        pltpu.async_copy(hs_hbm.at[sbuf.at[k, pl.ds(j * ECH, ECH)]],
                         rowsv.at[(g * GCH + j) % NRING],
                         gsem.at[(g * GCH + j) % NRING])

    def run(hs_hbm, out_hbm):
        zero_acc()
        fire_idx(0, 0)
        wait_idx(0)
        for j in range(NRING - 1):
            fire_gather(hs_hbm, 0, 0, j)

        def group(g, carry):
            k = lax.rem(g, 2)
            kn = lax.rem(g + 1, 2)

            @pl.when(g + 1 < NG)
            def _pf_idx():
                fire_idx(g + 1, kn)

            for j in range(GCH):
                kr = j % NRING
                pltpu.make_async_copy(
                    hs_hbm.at[sbuf.at[k, pl.ds(j * ECH, ECH)]],
                    rowsv.at[kr], gsem.at[kr]).wait()
                if j == 2:
                    @pl.when(g + 1 < NG)
                    def _w_idx():
                        wait_idx(kn)
                nxt = j + NRING - 1
                if nxt < GCH:
                    fire_gather(hs_hbm, k, g, nxt)
                else:
                    @pl.when(g + 1 < NG)
                    def _pf_gather():
                        fire_gather(hs_hbm, kn, g + 1, nxt - GCH)
                pltpu.sync_copy(rowsv.at[kr],
                                accsp.at[dbuf.at[k, pl.ds(j * ECH, ECH)]],
                                add=True)
            return carry

        lax.fori_loop(0, NG, group, 0)
        writeback(out_hbm)

    def run_deg(out_hbm):
        zero_acc()
        fire_idx(0, 0)

        def group(g, carry):
            k = lax.rem(g, 2)
            kn = lax.rem(g + 1, 2)
            wait_idx(k)

            @pl.when(g + 1 < NG)
            def _pf_idx():
                fire_idx(g + 1, kn)

            for j in range(GCH):
                pltpu.sync_copy(onesv,
                                accsp.at[dbuf.at[k, pl.ds(j * ECH, ECH)]],
                                add=True)
            return carry

        lax.fori_loop(0, NG, group, 0)
        writeback(out_hbm)

    @pl.when(jnp.logical_and(c == 0, is_deg))
    def _deg():
        run_deg(ast_hbm.at[0])

    @pl.when(jnp.logical_and(c == 0, jnp.logical_not(is_deg)))
    def _c0():
        run(hst_hbm.at[0], ast_hbm.at[0])
        run(hst_hbm.at[1], ast_hbm.at[1])

    @pl.when(jnp.logical_and(c == 1, jnp.logical_not(is_deg)))
    def _c1():
        run(hst_hbm.at[2], ast_hbm.at[2])
        run(hst_hbm.at[3], ast_hbm.at[3])


_PSHAPE = jax.ShapeDtypeStruct((NSUB, NGRAPH, HQ), jnp.float32)


@functools.partial(
    pl.kernel,
    out_type=jax.ShapeDtypeStruct((4, NSUB, NGRAPH, HQ), jnp.float32),
    mesh=_mesh,
    compiler_params=_SC_PARAMS,
    scratch_types=[
        pltpu.VMEM((SR, HQ), jnp.float32),
        pltpu.VMEM((RT + 16,), jnp.int32),
        pltpu.VMEM((NGRAPH, HQ), jnp.float32),
    ],
)
def _segmax_kernel(hst_hbm, batch_hbm, ost_hbm, hv, bv, tbl):
    c = lax.axis_index("c")
    s = lax.axis_index("s")
    r0 = s * RT
    nvalid = jnp.where(s == NSUB - 1, N - (NSUB - 1) * RT, RT)
    pltpu.sync_copy(batch_hbm.at[pl.ds(r0, RT)], bv.at[pl.ds(0, RT)])

    iota = lax.iota(jnp.int32, 16)
    neginf = jnp.full((16,), -jnp.inf, jnp.float32)

    def run(h_hbm, out_hbm):
        def init(i, carry):
            plsc.store_scatter(tbl, [jnp.full((16,), i, jnp.int32), iota],
                               neginf)
            return carry

        lax.fori_loop(0, NGRAPH, init, 0)

        for ci in range(RT // SR):
            pltpu.sync_copy(h_hbm.at[pl.ds(r0 + ci * SR, SR)], hv)
            nv = jnp.clip(nvalid - ci * SR, 0, SR)

            def step(i, carry):
                b = bv[pl.ds(ci * SR + i, 16)][0]
                bi = jnp.full((16,), b, jnp.int32)
                ii = jnp.full((16,), i, jnp.int32)
                rv = plsc.load_gather(hv, [ii, iota])
                tv = plsc.load_gather(tbl, [bi, iota])
                plsc.store_scatter(tbl, [bi, iota], jnp.maximum(rv, tv))
                return carry

            lax.fori_loop(0, nv, step, 0)
        pltpu.sync_copy(tbl, out_hbm.at[s])

    @pl.when(c == 0)
    def _c0():
        run(hst_hbm.at[0], ost_hbm.at[0])
        run(hst_hbm.at[1], ost_hbm.at[1])

    @pl.when(c == 1)
    def _c1():
        run(hst_hbm.at[2], ost_hbm.at[2])
        run(hst_hbm.at[3], ost_hbm.at[3])


_RB = 2560
_NB = NP // _RB

_PREC = jax.lax.Precision.DEFAULT


def _mid_body(flags_ref, a_ref, s_ref, xq_ref,
              d_ref, b_ref, w_ref, o_ref, dout_ref):
    is_deg = flags_ref[0] == 1
    use_t = flags_ref[1] == 1
    use_relu = flags_ref[2] == 1
    use_scale = flags_ref[3] == 1

    a = a_ref[...]
    sv = s_ref[...]
    b = b_ref[...]
    d = jnp.where(is_deg, lax.rsqrt(jnp.abs(a[0, :, 0:1]) + 1.0),
                  d_ref[...][:, 0:1])
    w = w_ref[...]
    z = None
    for q in range(4):
        base = xq_ref[...] if q == 0 else 0.0
        tq = jnp.where(use_t, d * (a[q] + sv[q]) + b[:, q*HQ:(q+1)*HQ],
                       base)
        hq = jnp.where(use_relu, jnp.maximum(tq, 0.0), tq)
        zq = jnp.dot(hq, w[q*HQ:(q+1)*HQ], precision=_PREC)
        z = zq if z is None else z + zq
    scale = jnp.where(use_scale, d, 1.0)
    zs = scale * z
    o_ref[...] = jnp.stack(
        [zs[:, 0:HQ], zs[:, HQ:2*HQ], zs[:, 2*HQ:3*HQ], zs[:, 3*HQ:]])
    dout_ref[...] = jnp.broadcast_to(d, (_RB, HQ))


def _mid_call(flags, a, s, xq, d, b, W):
    qspec = pl.BlockSpec((4, _RB, HQ), lambda i: (0, i, 0))
    return pl.pallas_call(
        _mid_body,
        grid=(_NB,),
        in_specs=[
            pl.BlockSpec(memory_space=pltpu.SMEM),
            qspec,
            qspec,
            pl.BlockSpec((_RB, HQ), lambda i: (i, 0)),
            pl.BlockSpec((_RB, HQ), lambda i: (i, 0)),
            pl.BlockSpec((1, 64), lambda i: (0, 0)),
            pl.BlockSpec((64, 64), lambda i: (0, 0)),
        ],
        out_specs=[qspec, pl.BlockSpec((_RB, HQ), lambda i: (i, 0))],
        out_shape=[jax.ShapeDtypeStruct((4, NP, HQ), jnp.float32),
                   jax.ShapeDtypeStruct((NP, HQ), jnp.float32)],
    )(flags, a, s, xq, d, b, W)


def _readout_body(p_ref, wl_ref, bl_ref, out_ref):
    wl = wl_ref[...]
    p = p_ref[...]
    z = None
    for q in range(4):
        gq = jnp.max(p[q], axis=0)
        zq = jnp.dot(gq, wl[q*HQ:(q+1)*HQ], precision=_PREC)
        z = zq if z is None else z + zq
    out_ref[...] = z + bl_ref[...]


def _readout_call(p, Wl, bl):
    return pl.pallas_call(
        _readout_body,
        out_shape=jax.ShapeDtypeStruct((NGRAPH, 10), jnp.float32),
    )(p, Wl, bl)


def kernel(x, edge_index, batch, W1, b1, W2, b2, W3, b3, Wl, bl):
    i32 = jnp.int32
    f32 = jnp.float32
    pad = jnp.full((EP - E,), NP - 1, i32)
    src1 = jnp.concatenate([edge_index[0].astype(i32), pad])
    dst1 = jnp.concatenate([edge_index[1].astype(i32), pad])
    xq = jnp.zeros((NP, HQ), f32).at[:N, :x.shape[1]].set(x)
    batchp = jnp.concatenate(
        [batch.astype(i32), jnp.full((NP - N,), NGRAPH - 1, i32)])
    z16 = jnp.zeros((ZR, HQ), f32)

    w1p = jnp.zeros((64, 64), f32).at[:x.shape[1]].set(W1)
    eye = jnp.eye(64, dtype=f32)
    Ws = jnp.stack([w1p, W2, W3, eye])
    bs = jnp.stack([jnp.zeros((1, 64), f32), b1.reshape(1, 64),
                    b2.reshape(1, 64), b3.reshape(1, 64)])
    flags = jnp.array([[1, 0, 0, 1],
                       [0, 1, 1, 1],
                       [0, 1, 1, 1],
                       [0, 1, 0, 0]], dtype=i32)
    flags_sc = jnp.zeros((4, 16), i32).at[0, 0].set(1)
    ones_rows = jnp.ones((ECH, HQ), f32)

    ones_st = jnp.ones((4, NP, HQ), f32)
    d0 = jnp.ones((NP, HQ), f32)

    def body(k, carry):
        hst, d = carry
        fl = lax.dynamic_index_in_dim(flags, k, 0, keepdims=False)
        fls = lax.dynamic_index_in_dim(flags_sc, k, 0, keepdims=False)
        W = lax.dynamic_index_in_dim(Ws, k, 0, keepdims=False)
        b = lax.dynamic_index_in_dim(bs, k, 0, keepdims=False)
        ast = _scatter_kernel(hst, src1, dst1, z16, ones_rows, fls)
        hst, d = _mid_call(fl, ast, hst, xq, d, b, W)
        return (hst, d)

    n_pass = lax.optimization_barrier(jnp.int32(4))
    hst, d = lax.fori_loop(0, n_pass, body, (ones_st, d0))

    p = _segmax_kernel(hst, batchp)
    return _readout_call(p, Wl, bl.reshape(1, 10))
